# baseline (speedup 1.0000x reference)
import functools
import jax
import jax.numpy as jnp
from jax import lax
from jax.experimental import pallas as pl
from jax.experimental.pallas import tpu as pltpu
from jax.experimental.pallas import tpu_sc as plsc

D_MODEL = 64
ROWS = 4096
COLS = 200
NC = 2
NS = 16
NW = NC * NS
IPW = ROWS // NW
NBUF = 4
SCALE = 8.0
L_SHAPE = (COLS, D_MODEL // 8, NW, 8, 128)


def _embed_body(x_hbm, table_hbm, out_hbm, idx_v, idx_t, gbufs, tbufs, gsems, ssems):
    wid = lax.axis_index("s") * NC + lax.axis_index("c")
    i0 = wid * IPW

    pltpu.sync_copy(x_hbm.at[pl.ds(i0, IPW)], idx_v)

    iota16 = lax.broadcasted_iota(jnp.int32, (16,), 0)

    @plsc.parallel_loop(0, COLS, 1, unroll=2)
    def _(j):
        cols = jnp.broadcast_to(j, (16,)).astype(jnp.int32)
        for blk in range(IPW // 16):
            rows = blk * 16 + iota16
            idx_t[j, pl.ds(blk * 16, 16)] = plsc.load_gather(idx_v, [rows, cols])

    def fire_gather(j, b):
        pltpu.async_copy(table_hbm.at[idx_t.at[j]], gbufs[b], gsems[b])

    def wait_gather(b):
        pltpu.make_async_copy(table_hbm.at[idx_t.at[0]], gbufs[b], gsems[b]).wait()

    def fire_store(j, b):
        pltpu.async_copy(tbufs[b], out_hbm.at[j, :, wid], ssems[b])

    def wait_store(b):
        pltpu.make_async_copy(tbufs[b], out_hbm.at[0, :, 0], ssems[b]).wait()

    def tscale(b):
        @plsc.parallel_loop(0, D_MODEL, 1, unroll=2)
        def _(d):
            dt = d // 8
            ds_ = d % 8
            cols = jnp.broadcast_to(d, (16,)).astype(jnp.int32)
            for blk in range(IPW // 16):
                rows = blk * 16 + iota16
                vals = plsc.load_gather(gbufs[b], [rows, cols])
                tbufs[b][dt, ds_, pl.ds(blk * 16, 16)] = vals * SCALE

    for b in range(NBUF):
        fire_gather(b, b)

    for b in range(NBUF):
        wait_gather(b)
        tscale(b)
        fire_store(b, b)
        fire_gather(b + NBUF, b)

    def outer(o, carry):
        j0 = o * NBUF
        for b in range(NBUF):
            j = j0 + b
            wait_gather(b)
            wait_store(b)
            tscale(b)
            fire_store(j, b)

            @pl.when(j + NBUF < COLS)
            def _():
                fire_gather(j + NBUF, b)

        return carry

    lax.fori_loop(1, COLS // NBUF, outer, 0)

    for b in range(NBUF):
        wait_store(b)


def _embed_builder(interpret=False):
    mesh = plsc.VectorSubcoreMesh(core_axis_name="c", subcore_axis_name="s")
    return functools.partial(
        pl.kernel,
        mesh=mesh,
        out_type=jax.ShapeDtypeStruct(L_SHAPE, jnp.float32),
        scratch_types=[
            pltpu.VMEM((IPW, COLS), jnp.int32),
            pltpu.VMEM((COLS, IPW), jnp.int32),
            [pltpu.VMEM((IPW, D_MODEL), jnp.float32) for _ in range(NBUF)],
            [pltpu.VMEM((D_MODEL // 8, 8, 128), jnp.float32) for _ in range(NBUF)],
            [pltpu.SemaphoreType.DMA for _ in range(NBUF)],
            [pltpu.SemaphoreType.DMA for _ in range(NBUF)],
        ],
        compiler_params=pltpu.CompilerParams(
            use_tc_tiling_on_sc=False, needs_layout_passes=False
        ),
        interpret=interpret,
    )(_embed_body)


_embed = _embed_builder()


def kernel(x, table):
    l = _embed(x.astype(jnp.int32), table)
    return jnp.transpose(l, (2, 4, 0, 1, 3)).reshape(ROWS, COLS, D_MODEL)


# speedup vs baseline: 1.4049x; 1.4049x over previous
import functools
import jax
import jax.numpy as jnp
from jax import lax
from jax.experimental import pallas as pl
from jax.experimental.pallas import tpu as pltpu
from jax.experimental.pallas import tpu_sc as plsc

D_MODEL = 64
ROWS = 4096
COLS = 200
VOCAB = 1000000
NC = 2
NS = 16
NW = NC * NS
IPW = ROWS // NW
NBUF = 4
SCALE = 8.0
L_SHAPE = (COLS, D_MODEL // 8, NW, 8, 128)
NBLK = IPW // 16

NT = (VOCAB // 128)
NT_W = 246
PAIRS = VOCAB // 2


def _repack_body(tt_hbm, tail_hbm, t2_hbm, inbufs, outbufs, tailbuf, isems, osems):
    wid = lax.axis_index("s") * NC + lax.axis_index("c")
    vt0 = wid * 244 + lax.min(wid, 4)
    ntw = 244 + jnp.where(wid < 4, 1, 0)

    iota = lax.broadcasted_iota(jnp.int32, (16,), 0)
    zero16 = iota * 0
    rots = [lax.bitwise_and(iota + p, 15) for p in range(16)]
    dsrc = [rots[p] * 128 + iota for p in range(16)]
    ddst = [iota * 64 + rots[p] for p in range(16)]

    def vt_of(k):
        return lax.min(vt0 + lax.min(k, ntw - 1), NT - 1)

    def fire_in(k, b):
        vt = vt_of(k)
        pltpu.async_copy(
            tt_hbm.at[:, pl.ds(vt * 128, 128)], inbufs[b], isems[b]
        )

    def wait_in(b):
        pltpu.make_async_copy(
            tt_hbm.at[:, pl.ds(0, 128)], inbufs[b], isems[b]
        ).wait()

    def fire_out(k, b):
        vt = vt_of(k)
        pltpu.async_copy(outbufs[b], t2_hbm.at[pl.ds(vt * 64, 64)], osems[b])

    def wait_out(b):
        pltpu.make_async_copy(
            outbufs[b], t2_hbm.at[pl.ds(0, 64)], osems[b]
        ).wait()

    def transpose_slab(b):
        @plsc.parallel_loop(0, 32, 1)
        def _(blk):
            d0 = lax.shift_left(lax.shift_right_logical(blk, 3), 4)
            v0 = lax.shift_left(lax.bitwise_and(blk, 7), 4)
            sbase = jnp.broadcast_to(d0 * 128 + v0, (16,)).astype(jnp.int32)
            dbase = jnp.broadcast_to(v0 * 64 + d0, (16,)).astype(jnp.int32)
            for p in range(16):
                vals = plsc.load_gather(inbufs[b], [zero16, sbase + dsrc[p]])
                plsc.store_scatter(
                    outbufs[b], [zero16, dbase + ddst[p]], vals * SCALE
                )

    fire_in(0, 0)
    fire_in(1, 1)
    wait_in(0)
    transpose_slab(0)
    fire_out(0, 0)
    fire_in(2, 0)
    wait_in(1)
    transpose_slab(1)
    fire_out(1, 1)
    fire_in(3, 1)

    def outer(o, carry):
        for b in range(2):
            k = 2 * o + b
            wait_in(b)
            wait_out(b)
            transpose_slab(b)
            fire_out(k, b)

            @pl.when(k + 2 < NT_W)
            def _():
                fire_in(k + 2, b)

        return carry

    lax.fori_loop(1, NT_W // 2, outer, 0)
    wait_out(0)
    wait_out(1)

    @pl.when(wid == NW - 1)
    def _():
        pltpu.sync_copy(tail_hbm, tailbuf)

        @plsc.parallel_loop(0, 32, 1)
        def _(r):
            for c in range(8):
                sl = pl.ds(c * 16, 16)
                tailbuf[r, sl] = tailbuf[r, sl] * SCALE

        pltpu.sync_copy(tailbuf, t2_hbm.at[pl.ds(PAIRS - 32, 32)])


def _repack_builder():
    mesh = plsc.VectorSubcoreMesh(core_axis_name="c", subcore_axis_name="s")
    return functools.partial(
        pl.kernel,
        mesh=mesh,
        out_type=jax.ShapeDtypeStruct((PAIRS, 128), jnp.float32),
        scratch_types=[
            [pltpu.VMEM((D_MODEL, 128), jnp.float32) for _ in range(2)],
            [pltpu.VMEM((D_MODEL, 128), jnp.float32) for _ in range(2)],
            pltpu.VMEM((32, 128), jnp.float32),
            [pltpu.SemaphoreType.DMA for _ in range(2)],
            [pltpu.SemaphoreType.DMA for _ in range(2)],
        ],
        compiler_params=pltpu.CompilerParams(
            use_tc_tiling_on_sc=True, needs_layout_passes=False
        ),
    )(_repack_body)


def _lookup_body(
    xw_hbm, t2_hbm, out_hbm, idx_t, i2bufs, offbufs, gbufs, tbufs, gsems, ssems
):
    wid = lax.axis_index("s") * NC + lax.axis_index("c")

    pltpu.sync_copy(xw_hbm.at[wid], idx_t)

    iota = lax.broadcasted_iota(jnp.int32, (16,), 0)
    zero16 = iota * 0
    i128 = iota * 128

    def prep_idx(j, b):
        for blk in range(NBLK):
            sl = pl.ds(blk * 16, 16)
            raw = idx_t[j, sl]
            i2bufs[b][sl] = lax.shift_right_logical(raw, 1)
            offbufs[b][sl] = lax.shift_left(lax.bitwise_and(raw, 1), 6)

    def fire_gather(b):
        pltpu.async_copy(t2_hbm.at[i2bufs[b]], gbufs[b], gsems[b])

    def wait_gather(b):
        pltpu.make_async_copy(t2_hbm.at[i2bufs[b]], gbufs[b], gsems[b]).wait()

    def fire_store(j, b):
        pltpu.async_copy(tbufs[b], out_hbm.at[j, :, wid], ssems[b])

    def wait_store(b):
        pltpu.make_async_copy(tbufs[b], out_hbm.at[0, :, 0], ssems[b]).wait()

    def tscale(b):
        fbases = []
        for blk in range(NBLK):
            sl = pl.ds(blk * 16, 16)
            fbases.append(offbufs[b][sl] + (blk * 16 * 128 + i128))

        @plsc.parallel_loop(0, D_MODEL, 1, unroll=4)
        def _(d):
            dt = lax.shift_right_logical(d, 3)
            ds_ = lax.bitwise_and(d, 7)
            for blk in range(NBLK):
                vals = plsc.load_gather(gbufs[b], [zero16, fbases[blk] + d])
                tbufs[b][dt, ds_, pl.ds(blk * 16, 16)] = vals

    for b in range(NBUF):
        prep_idx(b, b)
        fire_gather(b)

    for b in range(NBUF):
        wait_gather(b)
        tscale(b)
        fire_store(b, b)
        prep_idx(b + NBUF, b)
        fire_gather(b)

    def outer(o, carry):
        j0 = o * NBUF
        for b in range(NBUF):
            j = j0 + b
            wait_gather(b)
            wait_store(b)
            tscale(b)
            fire_store(j, b)

            @pl.when(j + NBUF < COLS)
            def _():
                prep_idx(j + NBUF, b)
                fire_gather(b)

        return carry

    lax.fori_loop(1, COLS // NBUF, outer, 0)

    for b in range(NBUF):
        wait_store(b)


def _lookup_builder():
    mesh = plsc.VectorSubcoreMesh(core_axis_name="c", subcore_axis_name="s")
    return functools.partial(
        pl.kernel,
        mesh=mesh,
        out_type=jax.ShapeDtypeStruct(L_SHAPE, jnp.float32),
        scratch_types=[
            pltpu.VMEM((COLS, IPW), jnp.int32),
            [pltpu.VMEM((IPW,), jnp.int32) for _ in range(NBUF)],
            [pltpu.VMEM((IPW,), jnp.int32) for _ in range(NBUF)],
            [pltpu.VMEM((IPW, 128), jnp.float32) for _ in range(NBUF)],
            [pltpu.VMEM((D_MODEL // 8, 8, 128), jnp.float32) for _ in range(NBUF)],
            [pltpu.SemaphoreType.DMA for _ in range(NBUF)],
            [pltpu.SemaphoreType.DMA for _ in range(NBUF)],
        ],
        compiler_params=pltpu.CompilerParams(
            use_tc_tiling_on_sc=False, needs_layout_passes=False
        ),
    )(_lookup_body)


_repack = _repack_builder()
_lookup = _lookup_builder()


def kernel(x, table):
    xw = x.astype(jnp.int32).reshape(NW, IPW, COLS).transpose(0, 2, 1)
    tt = table.T
    tail = lax.slice(table, (VOCAB - 64, 0), (VOCAB, D_MODEL)).reshape(32, 128)
    t2 = _repack(tt, tail)
    l = _lookup(xw, t2)
    return jnp.transpose(l, (2, 4, 0, 1, 3)).reshape(ROWS, COLS, D_MODEL)


# speedup vs baseline: 1.4283x; 1.0167x over previous
import functools
import jax
import jax.numpy as jnp
from jax import lax
from jax.experimental import pallas as pl
from jax.experimental.pallas import tpu as pltpu
from jax.experimental.pallas import tpu_sc as plsc

D_MODEL = 64
ROWS = 4096
COLS = 200
VOCAB = 1000000
NC = 2
NS = 16
NW = NC * NS
IPW = ROWS // NW
NBUF = 4
SCALE = 8.0
L_SHAPE = (COLS, D_MODEL // 8, NW, 8, 128)
NBLK = IPW // 16

NT = (VOCAB // 128)
NT_W = 246
PAIRS = VOCAB // 2


def _repack_body(tt_hbm, tail_hbm, t2_hbm, inbufs, outbufs, tailbuf, isems, osems):
    wid = lax.axis_index("s") * NC + lax.axis_index("c")
    vt0 = wid * 244 + lax.min(wid, 4)
    ntw = 244 + jnp.where(wid < 4, 1, 0)

    iota = lax.broadcasted_iota(jnp.int32, (16,), 0)
    zero16 = iota * 0
    rots = [lax.bitwise_and(iota + p, 15) for p in range(16)]
    dsrc = [rots[p] * 128 + iota for p in range(16)]
    ddst = [iota * 64 + rots[p] for p in range(16)]

    def vt_of(k):
        return lax.min(vt0 + lax.min(k, ntw - 1), NT - 1)

    def fire_in(k, b):
        vt = vt_of(k)
        pltpu.async_copy(
            tt_hbm.at[:, pl.ds(vt * 128, 128)], inbufs[b], isems[b]
        )

    def wait_in(b):
        pltpu.make_async_copy(
            tt_hbm.at[:, pl.ds(0, 128)], inbufs[b], isems[b]
        ).wait()

    def fire_out(k, b):
        vt = vt_of(k)
        pltpu.async_copy(outbufs[b], t2_hbm.at[pl.ds(vt * 64, 64)], osems[b])

    def wait_out(b):
        pltpu.make_async_copy(
            outbufs[b], t2_hbm.at[pl.ds(0, 64)], osems[b]
        ).wait()

    def transpose_slab(b):
        @plsc.parallel_loop(0, 32, 1)
        def _(blk):
            d0 = lax.shift_left(lax.shift_right_logical(blk, 3), 4)
            v0 = lax.shift_left(lax.bitwise_and(blk, 7), 4)
            sbase = jnp.broadcast_to(d0 * 128 + v0, (16,)).astype(jnp.int32)
            dbase = jnp.broadcast_to(v0 * 64 + d0, (16,)).astype(jnp.int32)
            for p in range(16):
                vals = plsc.load_gather(inbufs[b], [zero16, sbase + dsrc[p]])
                plsc.store_scatter(
                    outbufs[b], [zero16, dbase + ddst[p]], vals * SCALE
                )

    fire_in(0, 0)
    fire_in(1, 1)
    wait_in(0)
    transpose_slab(0)
    fire_out(0, 0)
    fire_in(2, 0)
    wait_in(1)
    transpose_slab(1)
    fire_out(1, 1)
    fire_in(3, 1)

    def outer(o, carry):
        for b in range(2):
            k = 2 * o + b
            wait_in(b)
            wait_out(b)
            transpose_slab(b)
            fire_out(k, b)

            @pl.when(k + 2 < NT_W)
            def _():
                fire_in(k + 2, b)

        return carry

    lax.fori_loop(1, NT_W // 2, outer, 0)
    wait_out(0)
    wait_out(1)

    @pl.when(wid == NW - 1)
    def _():
        pltpu.sync_copy(tail_hbm, tailbuf)

        @plsc.parallel_loop(0, 32, 1)
        def _(r):
            for c in range(8):
                sl = pl.ds(c * 16, 16)
                tailbuf[r, sl] = tailbuf[r, sl] * SCALE

        pltpu.sync_copy(tailbuf, t2_hbm.at[pl.ds(PAIRS - 32, 32)])


def _repack_builder():
    mesh = plsc.VectorSubcoreMesh(core_axis_name="c", subcore_axis_name="s")
    return functools.partial(
        pl.kernel,
        mesh=mesh,
        out_type=jax.ShapeDtypeStruct((PAIRS, 128), jnp.float32),
        scratch_types=[
            [pltpu.VMEM((D_MODEL, 128), jnp.float32) for _ in range(2)],
            [pltpu.VMEM((D_MODEL, 128), jnp.float32) for _ in range(2)],
            pltpu.VMEM((32, 128), jnp.float32),
            [pltpu.SemaphoreType.DMA for _ in range(2)],
            [pltpu.SemaphoreType.DMA for _ in range(2)],
        ],
        compiler_params=pltpu.CompilerParams(
            use_tc_tiling_on_sc=True, needs_layout_passes=False
        ),
    )(_repack_body)


def _lookup_body(xw_hbm, t2_hbm, out_hbm, idx_t, gbufs, tbufs, gsems, ssems):
    wid = lax.axis_index("s") * NC + lax.axis_index("c")

    pltpu.sync_copy(xw_hbm.at[wid], idx_t)

    iota = lax.broadcasted_iota(jnp.int32, (16,), 0)
    zero16 = iota * 0
    fbases = [(blk * 16 * D_MODEL) + iota * D_MODEL for blk in range(NBLK)]

    def fire_gather(j, b):
        pltpu.async_copy(t2_hbm.at[idx_t.at[j]], gbufs[b], gsems[b])

    def wait_gather(b):
        pltpu.make_async_copy(t2_hbm.at[idx_t.at[0]], gbufs[b], gsems[b]).wait()

    def fire_store(j, b):
        pltpu.async_copy(tbufs[b], out_hbm.at[j, :, wid], ssems[b])

    def wait_store(b):
        pltpu.make_async_copy(tbufs[b], out_hbm.at[0, :, 0], ssems[b]).wait()

    def tscale(b):
        @plsc.parallel_loop(0, D_MODEL, 1, unroll=4)
        def _(d):
            dt = lax.shift_right_logical(d, 3)
            ds_ = lax.bitwise_and(d, 7)
            for blk in range(NBLK):
                vals = plsc.load_gather(gbufs[b], [zero16, fbases[blk] + d])
                tbufs[b][dt, ds_, pl.ds(blk * 16, 16)] = vals

    for b in range(NBUF):
        fire_gather(b, b)

    for b in range(NBUF):
        wait_gather(b)
        tscale(b)
        fire_store(b, b)
        fire_gather(b + NBUF, b)

    def outer(o, carry):
        j0 = o * NBUF
        for b in range(NBUF):
            j = j0 + b
            wait_gather(b)
            wait_store(b)
            tscale(b)
            fire_store(j, b)

            @pl.when(j + NBUF < COLS)
            def _():
                fire_gather(j + NBUF, b)

        return carry

    lax.fori_loop(1, COLS // NBUF, outer, 0)

    for b in range(NBUF):
        wait_store(b)


def _lookup_builder():
    mesh = plsc.VectorSubcoreMesh(core_axis_name="c", subcore_axis_name="s")
    return functools.partial(
        pl.kernel,
        mesh=mesh,
        out_type=jax.ShapeDtypeStruct(L_SHAPE, jnp.float32),
        scratch_types=[
            pltpu.VMEM((COLS, IPW), jnp.int32),
            [pltpu.VMEM((IPW, D_MODEL), jnp.float32) for _ in range(NBUF)],
            [pltpu.VMEM((D_MODEL // 8, 8, 128), jnp.float32) for _ in range(NBUF)],
            [pltpu.SemaphoreType.DMA for _ in range(NBUF)],
            [pltpu.SemaphoreType.DMA for _ in range(NBUF)],
        ],
        compiler_params=pltpu.CompilerParams(
            use_tc_tiling_on_sc=False, needs_layout_passes=False
        ),
    )(_lookup_body)


_repack = _repack_builder()
_lookup = _lookup_builder()


def kernel(x, table):
    xw = x.astype(jnp.int32).reshape(NW, IPW, COLS).transpose(0, 2, 1)
    tt = table.T
    tail = lax.slice(table, (VOCAB - 64, 0), (VOCAB, D_MODEL)).reshape(32, 128)
    t2 = _repack(tt, tail).reshape(VOCAB, D_MODEL)
    l = _lookup(xw, t2)
    return jnp.transpose(l, (2, 4, 0, 1, 3)).reshape(ROWS, COLS, D_MODEL)


# speedup vs baseline: 3.2984x; 2.3094x over previous
import functools
import jax
import jax.numpy as jnp
from jax import lax
from jax.experimental import pallas as pl
from jax.experimental.pallas import tpu as pltpu
from jax.experimental.pallas import tpu_sc as plsc

D_MODEL = 64
ROWS = 4096
COLS = 200
VOCAB = 1000000
NC = 2
NS = 16
NW = NC * NS
IPW = ROWS // NW
NBUF = 4
SCALE = 8.0
L_SHAPE = (COLS, D_MODEL // 8, NW, 8, 128)
NBLK = IPW // 16

NT = (VOCAB // 128)
NT_W = 246
PAIRS = VOCAB // 2


def _repack_body(tt_hbm, tail_hbm, t2_hbm, inbufs, outbufs, tailbuf, isems, osems):
    wid = lax.axis_index("s") * NC + lax.axis_index("c")
    vt0 = wid * 244 + lax.min(wid, 4)
    ntw = 244 + jnp.where(wid < 4, 1, 0)

    iota = lax.broadcasted_iota(jnp.int32, (16,), 0)
    zero16 = iota * 0
    rots = [lax.bitwise_and(iota + p, 15) for p in range(16)]
    dsrc = [rots[p] * 128 + iota for p in range(16)]
    ddst = [iota * 64 + rots[p] for p in range(16)]

    def vt_of(k):
        return lax.min(vt0 + lax.min(k, ntw - 1), NT - 1)

    def fire_in(k, b):
        vt = vt_of(k)
        pltpu.async_copy(
            tt_hbm.at[:, pl.ds(vt * 128, 128)], inbufs[b], isems[b]
        )

    def wait_in(b):
        pltpu.make_async_copy(
            tt_hbm.at[:, pl.ds(0, 128)], inbufs[b], isems[b]
        ).wait()

    def fire_out(k, b):
        vt = vt_of(k)
        pltpu.async_copy(outbufs[b], t2_hbm.at[pl.ds(vt * 64, 64)], osems[b])

    def wait_out(b):
        pltpu.make_async_copy(
            outbufs[b], t2_hbm.at[pl.ds(0, 64)], osems[b]
        ).wait()

    def transpose_slab(b):
        @plsc.parallel_loop(0, 32, 1)
        def _(blk):
            d0 = lax.shift_left(lax.shift_right_logical(blk, 3), 4)
            v0 = lax.shift_left(lax.bitwise_and(blk, 7), 4)
            sbase = jnp.broadcast_to(d0 * 128 + v0, (16,)).astype(jnp.int32)
            dbase = jnp.broadcast_to(v0 * 64 + d0, (16,)).astype(jnp.int32)
            for p in range(16):
                vals = plsc.load_gather(inbufs[b], [zero16, sbase + dsrc[p]])
                plsc.store_scatter(
                    outbufs[b], [zero16, dbase + ddst[p]], vals * SCALE
                )

    fire_in(0, 0)
    fire_in(1, 1)
    wait_in(0)
    transpose_slab(0)
    fire_out(0, 0)
    fire_in(2, 0)
    wait_in(1)
    transpose_slab(1)
    fire_out(1, 1)
    fire_in(3, 1)

    def outer(o, carry):
        for b in range(2):
            k = 2 * o + b
            wait_in(b)
            wait_out(b)
            transpose_slab(b)
            fire_out(k, b)

            @pl.when(k + 2 < NT_W)
            def _():
                fire_in(k + 2, b)

        return carry

    lax.fori_loop(1, NT_W // 2, outer, 0)
    wait_out(0)
    wait_out(1)

    @pl.when(wid == NW - 1)
    def _():
        pltpu.sync_copy(tail_hbm, tailbuf)

        @plsc.parallel_loop(0, 32, 1)
        def _(r):
            for c in range(8):
                sl = pl.ds(c * 16, 16)
                tailbuf[r, sl] = tailbuf[r, sl] * SCALE

        pltpu.sync_copy(tailbuf, t2_hbm.at[pl.ds(PAIRS - 32, 32)])


def _repack_builder():
    mesh = plsc.VectorSubcoreMesh(core_axis_name="c", subcore_axis_name="s")
    return functools.partial(
        pl.kernel,
        mesh=mesh,
        out_type=jax.ShapeDtypeStruct((PAIRS, 128), jnp.float32),
        scratch_types=[
            [pltpu.VMEM((D_MODEL, 128), jnp.float32) for _ in range(2)],
            [pltpu.VMEM((D_MODEL, 128), jnp.float32) for _ in range(2)],
            pltpu.VMEM((32, 128), jnp.float32),
            [pltpu.SemaphoreType.DMA for _ in range(2)],
            [pltpu.SemaphoreType.DMA for _ in range(2)],
        ],
        compiler_params=pltpu.CompilerParams(
            use_tc_tiling_on_sc=True, needs_layout_passes=False
        ),
    )(_repack_body)


def _lookup_body(xw_hbm, t2_hbm, out_hbm, idx_t, gbufs, tbufs, gsems, ssems):
    wid = lax.axis_index("s") * NC + lax.axis_index("c")

    pltpu.sync_copy(xw_hbm.at[wid], idx_t)

    iota = lax.broadcasted_iota(jnp.int32, (16,), 0)
    zero16 = iota * 0
    rots = [lax.bitwise_and(iota + p, 15) for p in range(16)]
    dsrc = [iota * D_MODEL + rots[p] for p in range(16)]
    ddst = [rots[p] * 128 + iota for p in range(16)]

    def fire_gather(j, b):
        pltpu.async_copy(t2_hbm.at[idx_t.at[j]], gbufs[b], gsems[b])

    def wait_gather(b):
        pltpu.make_async_copy(t2_hbm.at[idx_t.at[0]], gbufs[b], gsems[b]).wait()

    def fire_store(j, b):
        pltpu.async_copy(tbufs[b], out_hbm.at[j, :, wid], ssems[b])

    def wait_store(b):
        pltpu.make_async_copy(tbufs[b], out_hbm.at[0, :, 0], ssems[b]).wait()

    def tscale(b):
        @plsc.parallel_loop(0, 32, 1, unroll=2)
        def _(blk):
            d0 = lax.shift_left(lax.bitwise_and(blk, 3), 4)
            i0 = lax.shift_left(lax.shift_right_logical(blk, 2), 4)
            sbase = jnp.broadcast_to(i0 * D_MODEL + d0, (16,)).astype(jnp.int32)
            dbase = jnp.broadcast_to(d0 * 128 + i0, (16,)).astype(jnp.int32)
            for p in range(16):
                vals = plsc.load_gather(gbufs[b], [zero16, sbase + dsrc[p]])
                plsc.store_scatter(
                    tbufs[b], [zero16, zero16, dbase + ddst[p]], vals
                )

    for b in range(NBUF):
        fire_gather(b, b)

    for b in range(NBUF):
        wait_gather(b)
        tscale(b)
        fire_store(b, b)
        fire_gather(b + NBUF, b)

    def outer(o, carry):
        j0 = o * NBUF
        for b in range(NBUF):
            j = j0 + b
            wait_gather(b)
            wait_store(b)
            tscale(b)
            fire_store(j, b)

            @pl.when(j + NBUF < COLS)
            def _():
                fire_gather(j + NBUF, b)

        return carry

    lax.fori_loop(1, COLS // NBUF, outer, 0)

    for b in range(NBUF):
        wait_store(b)


def _lookup_builder():
    mesh = plsc.VectorSubcoreMesh(core_axis_name="c", subcore_axis_name="s")
    return functools.partial(
        pl.kernel,
        mesh=mesh,
        out_type=jax.ShapeDtypeStruct(L_SHAPE, jnp.float32),
        scratch_types=[
            pltpu.VMEM((COLS, IPW), jnp.int32),
            [pltpu.VMEM((IPW, D_MODEL), jnp.float32) for _ in range(NBUF)],
            [pltpu.VMEM((D_MODEL // 8, 8, 128), jnp.float32) for _ in range(NBUF)],
            [pltpu.SemaphoreType.DMA for _ in range(NBUF)],
            [pltpu.SemaphoreType.DMA for _ in range(NBUF)],
        ],
        compiler_params=pltpu.CompilerParams(
            use_tc_tiling_on_sc=False, needs_layout_passes=False
        ),
    )(_lookup_body)


_repack = _repack_builder()
_lookup = _lookup_builder()


def kernel(x, table):
    xw = x.astype(jnp.int32).reshape(NW, IPW, COLS).transpose(0, 2, 1)
    tt = table.T
    tail = lax.slice(table, (VOCAB - 64, 0), (VOCAB, D_MODEL)).reshape(32, 128)
    t2 = _repack(tt, tail).reshape(VOCAB, D_MODEL)
    l = _lookup(xw, t2)
    return jnp.transpose(l, (2, 4, 0, 1, 3)).reshape(ROWS, COLS, D_MODEL)
